# Initial kernel scaffold; baseline (speedup 1.0000x reference)
import functools

import jax
import jax.numpy as jnp
from jax import lax
from jax.experimental import pallas as pl
from jax.experimental.pallas import tpu as pltpu
from jax.experimental.pallas import tpu_sc as plsc

N = 10000
E = 320000
D = 128
EPS = 1e-9

NC = 2
NS = 16
NW = NC * NS
EPT = E // NW
K = 80
NCHUNK = EPT // K
RPT = N // NS


def _spmm_tile(x_hbm, edge_hbm, vals_hbm, zeros_hbm, out_hbm,
               src_v, dst_v, vals_v, rows_v, acc_sh, sem):
    c = lax.axis_index("c")
    s = lax.axis_index("s")
    wid = s * NC + c

    pltpu.sync_copy(zeros_hbm.at[pl.ds(s * RPT, RPT)],
                    acc_sh.at[pl.ds(s * RPT, RPT)])
    plsc.subcore_barrier()

    def chunk(j, carry):
        ebase = wid * EPT + j * K
        pltpu.sync_copy(edge_hbm.at[1, pl.ds(ebase, K)], src_v)
        pltpu.sync_copy(edge_hbm.at[0, pl.ds(ebase, K)], dst_v)
        pltpu.sync_copy(vals_hbm.at[pl.ds(ebase, K)], vals_v)
        pltpu.async_copy(x_hbm.at[src_v], rows_v, sem).wait()

        def scale(k, c2):
            v = vals_v[k]
            for dd in range(D // 16):
                sl = pl.ds(dd * 16, 16)
                rows_v[k, sl] = rows_v[k, sl] * v
            return c2
        lax.fori_loop(0, K, scale, 0, unroll=False)

        pltpu.sync_copy(rows_v, acc_sh.at[dst_v], add=True)
        return carry

    lax.fori_loop(0, NCHUNK, chunk, 0, unroll=False)

    plsc.subcore_barrier()
    pltpu.sync_copy(acc_sh.at[pl.ds(s * RPT, RPT)],
                    out_hbm.at[c, pl.ds(s * RPT, RPT)])


_spmm_call = pl.kernel(
    _spmm_tile,
    out_type=jax.ShapeDtypeStruct((NC, N, D), jnp.float32),
    mesh=plsc.VectorSubcoreMesh(core_axis_name="c", subcore_axis_name="s"),
    scratch_types=[
        pltpu.VMEM((K,), jnp.int32),
        pltpu.VMEM((K,), jnp.int32),
        pltpu.VMEM((K,), jnp.float32),
        pltpu.VMEM((K, D), jnp.float32),
        pltpu.VMEM_SHARED((N, D), jnp.float32),
        pltpu.SemaphoreType.DMA,
    ],
)


BLK = 2000


def _add2_body(p_ref, o_ref):
    o_ref[...] = p_ref[0] + p_ref[1]


def _add2(p):
    return pl.pallas_call(
        _add2_body,
        grid=(N // BLK,),
        in_specs=[pl.BlockSpec((NC, BLK, D), lambda i: (0, i, 0))],
        out_specs=pl.BlockSpec((BLK, D), lambda i: (i, 0)),
        out_shape=jax.ShapeDtypeStruct((N, D), jnp.float32),
    )(p)


def _f_nl(x, W, b, off, sca):
    vw = jnp.dot(x, W, preferred_element_type=jnp.float32) + b
    vw = jnp.maximum(vw, 0.0)
    mean = jnp.mean(vw, axis=1, keepdims=True)
    var = jnp.mean((vw - mean) ** 2, axis=1, keepdims=True)
    inv = lax.rsqrt(var + EPS)
    return (vw - mean) * inv * sca + off


def _dense_body(v_ref, s1_ref, p2_ref, w0_ref, w1_ref, w2_ref,
                b0_ref, b1_ref, b2_ref, of0_ref, of1_ref, of2_ref,
                sc0_ref, sc1_ref, sc2_ref, o_ref):
    s2 = p2_ref[0] + p2_ref[1]
    h0 = _f_nl(v_ref[...], w0_ref[...], b0_ref[...], of0_ref[...], sc0_ref[...])
    h1 = _f_nl(s1_ref[...], w1_ref[...], b1_ref[...], of1_ref[...], sc1_ref[...])
    h2 = _f_nl(s2, w2_ref[...], b2_ref[...], of2_ref[...], sc2_ref[...])
    o_ref[...] = h0 + h1 + h2


def _dense(vecs, s1, p2, W0, W1, W2, b0, b1, b2, off0, off1, off2,
           sca0, sca1, sca2):
    row_spec = pl.BlockSpec((BLK, D), lambda i: (i, 0))
    w_spec = pl.BlockSpec((D, D), lambda i: (0, 0))
    vec_spec = pl.BlockSpec((1, D), lambda i: (0, 0))
    return pl.pallas_call(
        _dense_body,
        grid=(N // BLK,),
        in_specs=[row_spec, row_spec,
                  pl.BlockSpec((NC, BLK, D), lambda i: (0, i, 0)),
                  w_spec, w_spec, w_spec,
                  vec_spec, vec_spec, vec_spec,
                  vec_spec, vec_spec, vec_spec,
                  vec_spec, vec_spec, vec_spec],
        out_specs=row_spec,
        out_shape=jax.ShapeDtypeStruct((N, D), jnp.float32),
    )(vecs, s1, p2, W0, W1, W2,
      b0.reshape(1, D), b1.reshape(1, D), b2.reshape(1, D),
      off0, off1, off2, sca0, sca1, sca2)


def kernel(vecs, edge_index, edge_vals, W0, W1, W2, b0, b1, b2,
           off0, off1, off2, sca0, sca1, sca2):
    zeros = jnp.zeros((N, D), jnp.float32)
    p1 = _spmm_call(vecs, edge_index, edge_vals, zeros)
    s1 = _add2(p1)
    p2 = _spmm_call(s1, edge_index, edge_vals, zeros)
    return _dense(vecs, s1, p2, W0, W1, W2, b0, b1, b2,
                  off0, off1, off2, sca0, sca1, sca2)


# speedup vs baseline: 4.2083x; 4.2083x over previous
import functools

import jax
import jax.numpy as jnp
from jax import lax
from jax.experimental import pallas as pl
from jax.experimental.pallas import tpu as pltpu
from jax.experimental.pallas import tpu_sc as plsc

N = 10000
E = 320000
D = 128
EPS = 1e-9

NC = 2
NS = 16
NW = NC * NS
EPT = E // NW
K = 80
NCHUNK = EPT // K
RPTW = 640


def _spmm_tile(x_hbm, src_hbm, dst_hbm, vals_hbm, zeros_hbm, out_hbm,
               src_v, dst_v, vals_v, rows_v, acc_sh, sem):
    c = lax.axis_index("c")
    s = lax.axis_index("s")
    wid = s * NC + c
    roff = jnp.minimum(s * RPTW, N - RPTW)

    pltpu.sync_copy(zeros_hbm, acc_sh.at[pl.ds(roff, RPTW)])
    plsc.subcore_barrier()

    def chunk(j, carry):
        ebase = wid * EPT + j * K
        pltpu.sync_copy(src_hbm.at[pl.ds(ebase, K)], src_v)
        pltpu.sync_copy(dst_hbm.at[pl.ds(ebase, K)], dst_v)
        pltpu.sync_copy(vals_hbm.at[pl.ds(ebase, K)], vals_v)
        pltpu.async_copy(x_hbm.at[src_v], rows_v, sem).wait()

        def scale(k16, c2):
            vv = vals_v[pl.ds(k16 * 16, 16)]
            for l in range(16):
                v = vv[l]
                row = k16 * 16 + l
                for dd in range(D // 16):
                    sl = pl.ds(dd * 16, 16)
                    rows_v[row, sl] = rows_v[row, sl] * v
            return c2
        lax.fori_loop(0, K // 16, scale, 0, unroll=False)

        pltpu.sync_copy(rows_v, acc_sh.at[dst_v], add=True)
        return carry

    lax.fori_loop(0, NCHUNK, chunk, 0, unroll=False)

    plsc.subcore_barrier()
    pltpu.sync_copy(acc_sh.at[pl.ds(roff, RPTW)],
                    out_hbm.at[c, pl.ds(roff, RPTW)])


_spmm_call = pl.kernel(
    _spmm_tile,
    out_type=jax.ShapeDtypeStruct((NC, N, D), jnp.float32),
    mesh=plsc.VectorSubcoreMesh(core_axis_name="c", subcore_axis_name="s"),
    scratch_types=[
        pltpu.VMEM((K,), jnp.int32),
        pltpu.VMEM((K,), jnp.int32),
        pltpu.VMEM((K,), jnp.float32),
        pltpu.VMEM((K, D), jnp.float32),
        pltpu.VMEM_SHARED((N, D), jnp.float32),
        pltpu.SemaphoreType.DMA,
    ],
)


def _spmm(x, src, dst, vals, zeros):
    return _spmm_call(x, src, dst, vals, zeros)


BLK = 2000


def _add2_body(p_ref, o_ref):
    o_ref[...] = p_ref[0] + p_ref[1]


def _add2(p):
    return pl.pallas_call(
        _add2_body,
        grid=(N // BLK,),
        in_specs=[pl.BlockSpec((NC, BLK, D), lambda i: (0, i, 0))],
        out_specs=pl.BlockSpec((BLK, D), lambda i: (i, 0)),
        out_shape=jax.ShapeDtypeStruct((N, D), jnp.float32),
    )(p)


def _f_nl(x, W, b, off, sca):
    vw = jnp.dot(x, W, preferred_element_type=jnp.float32) + b
    vw = jnp.maximum(vw, 0.0)
    mean = jnp.mean(vw, axis=1, keepdims=True)
    var = jnp.mean((vw - mean) ** 2, axis=1, keepdims=True)
    inv = lax.rsqrt(var + EPS)
    return (vw - mean) * inv * sca + off


def _dense_body(v_ref, s1_ref, p2_ref, w0_ref, w1_ref, w2_ref,
                b0_ref, b1_ref, b2_ref, of0_ref, of1_ref, of2_ref,
                sc0_ref, sc1_ref, sc2_ref, o_ref):
    s2 = p2_ref[0] + p2_ref[1]
    h0 = _f_nl(v_ref[...], w0_ref[...], b0_ref[...], of0_ref[...], sc0_ref[...])
    h1 = _f_nl(s1_ref[...], w1_ref[...], b1_ref[...], of1_ref[...], sc1_ref[...])
    h2 = _f_nl(s2, w2_ref[...], b2_ref[...], of2_ref[...], sc2_ref[...])
    o_ref[...] = h0 + h1 + h2


def _dense(vecs, s1, p2, W0, W1, W2, b0, b1, b2, off0, off1, off2,
           sca0, sca1, sca2):
    row_spec = pl.BlockSpec((BLK, D), lambda i: (i, 0))
    w_spec = pl.BlockSpec((D, D), lambda i: (0, 0))
    vec_spec = pl.BlockSpec((1, D), lambda i: (0, 0))
    return pl.pallas_call(
        _dense_body,
        grid=(N // BLK,),
        in_specs=[row_spec, row_spec,
                  pl.BlockSpec((NC, BLK, D), lambda i: (0, i, 0)),
                  w_spec, w_spec, w_spec,
                  vec_spec, vec_spec, vec_spec,
                  vec_spec, vec_spec, vec_spec,
                  vec_spec, vec_spec, vec_spec],
        out_specs=row_spec,
        out_shape=jax.ShapeDtypeStruct((N, D), jnp.float32),
    )(vecs, s1, p2, W0, W1, W2,
      b0.reshape(1, D), b1.reshape(1, D), b2.reshape(1, D),
      off0, off1, off2, sca0, sca1, sca2)


def kernel(vecs, edge_index, edge_vals, W0, W1, W2, b0, b1, b2,
           off0, off1, off2, sca0, sca1, sca2):
    zeros = jnp.zeros((RPTW, D), jnp.float32)
    dst = edge_index[0]
    src = edge_index[1]
    p1 = _spmm(vecs, src, dst, edge_vals, zeros)
    s1 = _add2(p1)
    p2 = _spmm(s1, src, dst, edge_vals, zeros)
    return _dense(vecs, s1, p2, W0, W1, W2, b0, b1, b2,
                  off0, off1, off2, sca0, sca1, sca2)
